# Initial kernel scaffold; baseline (speedup 1.0000x reference)
import jax
import jax.numpy as jnp
from jax.experimental import pallas as pl


def kernel(h, edge_index, W, b):
    raise NotImplementedError("write your pallas kernel here")


# speedup vs baseline: 6.5742x; 6.5742x over previous
import functools

import jax
import jax.numpy as jnp
from jax import lax
from jax.experimental import pallas as pl
from jax.experimental.pallas import tpu as pltpu
from jax.experimental.pallas import tpu_sc as plsc

N = 10000
E = 320000
D = 128

NC = 2
NS = 16
NW = NC * NS
EPW = E // NW
NPAD = 10240
CHUNK = 80
NCHUNK = EPW // CHUNK
NAGG = 10240
ROWS_PER_TILE = NAGG // NS
ZROWS = 128


def _hist_body(src_hbm, dst_hbm, out_hbm, src_v, dst_v, ho_v, hi_v):
    cid = lax.axis_index("c")
    sid = lax.axis_index("s")
    wid = sid * NC + cid
    base = wid * EPW

    zeros16 = jnp.zeros((16,), jnp.float32)

    def zloop(i, _):
        ho_v[pl.ds(i * 16, 16)] = zeros16
        hi_v[pl.ds(i * 16, 16)] = zeros16
        return 0

    lax.fori_loop(0, NPAD // 16, zloop, 0)

    pltpu.sync_copy(src_hbm.at[pl.ds(base, EPW)], src_v)
    pltpu.sync_copy(dst_hbm.at[pl.ds(base, EPW)], dst_v)

    ones16 = jnp.ones((16,), jnp.float32)

    def hloop(i, _):
        s = src_v[pl.ds(i * 16, 16)]
        d = dst_v[pl.ds(i * 16, 16)]
        plsc.addupdate_scatter(ho_v, [s], ones16)
        plsc.addupdate_scatter(hi_v, [d], ones16)
        return 0

    lax.fori_loop(0, EPW // 16, hloop, 0)

    pltpu.sync_copy(ho_v, out_hbm.at[wid, 0])
    pltpu.sync_copy(hi_v, out_hbm.at[wid, 1])


def _sc_hist(src, dst):
    mesh = plsc.VectorSubcoreMesh(core_axis_name="c", subcore_axis_name="s")
    return pl.kernel(
        _hist_body,
        out_type=jax.ShapeDtypeStruct((NW, 2, NPAD), jnp.float32),
        mesh=mesh,
        compiler_params=pltpu.CompilerParams(needs_layout_passes=False),
        scratch_types=[
            pltpu.VMEM((EPW,), jnp.int32),
            pltpu.VMEM((EPW,), jnp.int32),
            pltpu.VMEM((NPAD,), jnp.float32),
            pltpu.VMEM((NPAD,), jnp.float32),
        ],
    )(src, dst)


def _prescale_body(h_ref, hparts_ref, hs_ref):
    deg = jnp.sum(hparts_ref[...], axis=0)
    ns = lax.rsqrt(jnp.clip(deg, 1.0, None))
    hs_ref[...] = h_ref[...] * ns[:N, None]


def _tc_prescale(h, hparts):
    return pl.pallas_call(
        _prescale_body,
        out_shape=jax.ShapeDtypeStruct((N, D), jnp.float32),
    )(h, hparts)


def _gs_body(hs_hbm, src_hbm, dst_hbm, out_hbm, si_v, di_v, rows_v, zb_v, agg_sh, sem):
    cid = lax.axis_index("c")
    sid = lax.axis_index("s")
    wid = sid * NC + cid

    zeros16 = jnp.zeros((16,), jnp.float32)

    def zloop(i, _):
        r = i // (D // 16)
        c = i % (D // 16)
        zb_v[r, pl.ds(c * 16, 16)] = zeros16
        return 0

    lax.fori_loop(0, ZROWS * (D // 16), zloop, 0)

    row0 = sid * ROWS_PER_TILE

    def zcopy(k, _):
        pltpu.sync_copy(zb_v, agg_sh.at[pl.ds(row0 + k * ZROWS, ZROWS)])
        return 0

    lax.fori_loop(0, ROWS_PER_TILE // ZROWS, zcopy, 0)
    plsc.subcore_barrier()

    base = wid * EPW

    def eloop(c, _):
        off = base + c * CHUNK
        pltpu.sync_copy(src_hbm.at[pl.ds(off, CHUNK)], si_v)
        pltpu.sync_copy(dst_hbm.at[pl.ds(off, CHUNK)], di_v)
        pltpu.async_copy(hs_hbm.at[si_v], rows_v, sem).wait()
        pltpu.sync_copy(rows_v, agg_sh.at[di_v], add=True)
        return 0

    lax.fori_loop(0, NCHUNK, eloop, 0)
    plsc.subcore_barrier()

    def wloop(k, _):
        r = row0 + k * ZROWS
        pltpu.sync_copy(agg_sh.at[pl.ds(r, ZROWS)], zb_v)
        pltpu.sync_copy(zb_v, out_hbm.at[cid, pl.ds(r, ZROWS)])
        return 0

    lax.fori_loop(0, ROWS_PER_TILE // ZROWS, wloop, 0)


def _sc_gather_scatter(hs, src, dst):
    mesh = plsc.VectorSubcoreMesh(core_axis_name="c", subcore_axis_name="s")
    return pl.kernel(
        _gs_body,
        out_type=jax.ShapeDtypeStruct((NC, NAGG, D), jnp.float32),
        mesh=mesh,
        scratch_types=[
            pltpu.VMEM((CHUNK,), jnp.int32),
            pltpu.VMEM((CHUNK,), jnp.int32),
            pltpu.VMEM((CHUNK, D), jnp.float32),
            pltpu.VMEM((ZROWS, D), jnp.float32),
            pltpu.VMEM_SHARED((NAGG, D), jnp.float32),
            pltpu.SemaphoreType.DMA,
        ],
    )(hs, src, dst)


def _finish_body(agg_ref, hparts_ref, w_ref, b_ref, out_ref):
    deg = jnp.sum(hparts_ref[...], axis=0)
    nd = lax.rsqrt(jnp.clip(deg, 1.0, None))
    rst = (agg_ref[0, :N] + agg_ref[1, :N]) * nd[:N, None]
    rst = jnp.dot(rst, w_ref[...], preferred_element_type=jnp.float32)
    rst = rst + b_ref[...][None, :]
    out_ref[...] = jnp.where(rst > 0, rst, jnp.exp(jnp.minimum(rst, 0.0)) - 1.0)


def _tc_finish(agg, hparts, W, b):
    return pl.pallas_call(
        _finish_body,
        out_shape=jax.ShapeDtypeStruct((N, D), jnp.float32),
    )(agg, hparts, W, b)


@jax.jit
def kernel(h, edge_index, W, b):
    src = edge_index[0]
    dst = edge_index[1]
    hist = _sc_hist(src, dst)
    ho = hist[:, 0, :]
    hi = hist[:, 1, :]
    hs = _tc_prescale(h, ho)
    agg = _sc_gather_scatter(hs, src, dst)
    return _tc_finish(agg, hi, W, b)
